# baseline (speedup 1.0000x reference)
import jax
import jax.numpy as jnp
from jax import lax
from jax.experimental import pallas as pl
from jax.experimental.pallas import tpu as pltpu
from jax.experimental.pallas import tpu_sc as plsc

D_MODEL = 1024
BATCH = 128
L = 16

NC = 2
NS = 16
NWA = 16
BPW = BATCH // NWA


def _body(tok_hbm, idx_hbm, table_hbm, pos_hbm, out_hbm,
          tok_v, pidx_v, rows_v, pos_v, sem_t, sem_p):
    c = lax.axis_index("c")
    s = lax.axis_index("s")
    wid = s * NC + c

    @pl.when(wid < NWA)
    def _():
        base = wid * BPW

        cp_tok = pltpu.async_copy(tok_hbm.at[pl.ds(base, BPW)], tok_v, sem_t)
        cp_idx = pltpu.async_copy(idx_hbm, pidx_v.at[pl.ds(0, 1)], sem_p)
        cp_idx.wait()

        pidx_v[...] = pidx_v[...] + 1

        cp_p = pltpu.async_copy(pos_hbm.at[pidx_v.at[pl.ds(0, 1)]], pos_v, sem_p)
        cp_tok.wait()
        cp_t = pltpu.async_copy(table_hbm.at[tok_v], rows_v, sem_t)
        cp_p.wait()
        cp_t.wait()

        for b in range(BPW):
            def add_chunk(j, carry, b=b):
                off = j * (4 * L)
                for u in range(4):
                    pc = pos_v[0, pl.ds(off + u * L, L)]
                    rows_v[b, pl.ds(off + u * L, L)] += pc
                return carry

            lax.fori_loop(0, D_MODEL // (4 * L), add_chunk, 0)
            pltpu.async_copy(rows_v.at[pl.ds(b, 1)], out_hbm.at[base + b], sem_t)

        pltpu.make_async_copy(rows_v, out_hbm.at[pl.ds(0, BPW), 0], sem_t).wait()


def kernel(next_token, idx, speech_emb, pos_emb):
    mesh = plsc.VectorSubcoreMesh(
        core_axis_name="c", subcore_axis_name="s",
        num_cores=NC, num_subcores=NS)
    out = pl.kernel(
        _body,
        mesh=mesh,
        out_type=jax.ShapeDtypeStruct((BATCH, 1, D_MODEL), jnp.float32),
        scratch_types=[
            pltpu.VMEM((BPW,), jnp.int32),
            pltpu.VMEM((L,), jnp.int32),
            pltpu.VMEM((BPW, D_MODEL), jnp.float32),
            pltpu.VMEM((1, D_MODEL), jnp.float32),
            pltpu.SemaphoreType.DMA,
            pltpu.SemaphoreType.DMA,
        ],
        name="speech_embedding_sc",
    )(next_token.reshape(BATCH), idx, speech_emb, pos_emb)
    return out


# speedup vs baseline: 1.0576x; 1.0576x over previous
import jax
import jax.numpy as jnp
from jax import lax
from jax.experimental import pallas as pl
from jax.experimental.pallas import tpu as pltpu
from jax.experimental.pallas import tpu_sc as plsc

D_MODEL = 1024
BATCH = 128
L = 16

NC = 2
NS = 16
NW = NC * NS
BPW = BATCH // NW


def _body(tok_hbm, idx_hbm, table_hbm, pos_hbm, out_hbm,
          tok_v, pidx_v, rows_v, pos_v, sem_t, sem_p):
    c = lax.axis_index("c")
    s = lax.axis_index("s")
    wid = s * NC + c
    base = wid * BPW

    cp_tok = pltpu.async_copy(tok_hbm.at[wid], tok_v, sem_t)
    cp_idx = pltpu.async_copy(idx_hbm, pidx_v.at[pl.ds(0, 1)], sem_p)
    cp_idx.wait()

    pidx_v[...] = pidx_v[...] + 1

    cp_p = pltpu.async_copy(pos_hbm.at[pidx_v.at[pl.ds(0, 1)]], pos_v, sem_p)
    cp_tok.wait()
    cp_t = pltpu.async_copy(table_hbm.at[tok_v], rows_v, sem_t)
    cp_p.wait()
    cp_t.wait()

    for b in range(BPW):
        def add_chunk(j, carry, b=b):
            off = j * (4 * L)
            for u in range(4):
                pc = pos_v[0, pl.ds(off + u * L, L)]
                rows_v[b, pl.ds(off + u * L, L)] += pc
            return carry

        lax.fori_loop(0, D_MODEL // (4 * L), add_chunk, 0)
        pltpu.async_copy(rows_v.at[pl.ds(b, 1)], out_hbm.at[base + b], sem_t)

    pltpu.make_async_copy(rows_v, out_hbm.at[pl.ds(0, BPW), 0], sem_t).wait()


def kernel(next_token, idx, speech_emb, pos_emb):
    mesh = plsc.VectorSubcoreMesh(
        core_axis_name="c", subcore_axis_name="s",
        num_cores=NC, num_subcores=NS)
    out = pl.kernel(
        _body,
        mesh=mesh,
        out_type=jax.ShapeDtypeStruct((BATCH, 1, D_MODEL), jnp.float32),
        scratch_types=[
            pltpu.VMEM((BPW,), jnp.int32),
            pltpu.VMEM((L,), jnp.int32),
            pltpu.VMEM((BPW, D_MODEL), jnp.float32),
            pltpu.VMEM((1, D_MODEL), jnp.float32),
            pltpu.SemaphoreType.DMA,
            pltpu.SemaphoreType.DMA,
        ],
        name="speech_embedding_sc",
    )(next_token.reshape(NW, BPW), idx, speech_emb, pos_emb)
    return out


# speedup vs baseline: 1.1047x; 1.0446x over previous
import jax
import jax.numpy as jnp
from jax import lax
from jax.experimental import pallas as pl
from jax.experimental.pallas import tpu as pltpu
from jax.experimental.pallas import tpu_sc as plsc

D_MODEL = 1024
BATCH = 128
L = 16

NC = 2
NS = 16
NW = NC * NS
BPW = BATCH // NW


def _body(tok_hbm, idx_hbm, table_hbm, pos_hbm, out_hbm,
          tok_v, pidx_v, rows_v, pos_v, sem_t, sem_p):
    c = lax.axis_index("c")
    s = lax.axis_index("s")
    wid = s * NC + c
    base = wid * BPW

    cp_tok = pltpu.async_copy(tok_hbm.at[wid], tok_v, sem_t)
    cp_idx = pltpu.async_copy(idx_hbm, pidx_v.at[pl.ds(0, 1)], sem_p)
    cp_idx.wait()

    pidx_v[...] = pidx_v[...] + 1

    cp_p = pltpu.async_copy(pos_hbm.at[pidx_v.at[pl.ds(0, 1)]], pos_v, sem_p)
    cp_tok.wait()
    cp_t = pltpu.async_copy(table_hbm.at[tok_v], rows_v, sem_t)
    cp_p.wait()
    cp_t.wait()

    def add_chunk(j, carry):
        off = j * L
        pc = pos_v[0, pl.ds(off, L)]
        for b in range(BPW):
            rows_v[b, pl.ds(off, L)] += pc
        return carry

    lax.fori_loop(0, D_MODEL // L, add_chunk, 0)

    for b in range(BPW):
        pltpu.async_copy(rows_v.at[pl.ds(b, 1)], out_hbm.at[base + b], sem_t)
    pltpu.make_async_copy(rows_v, out_hbm.at[pl.ds(0, BPW), 0], sem_t).wait()


def kernel(next_token, idx, speech_emb, pos_emb):
    mesh = plsc.VectorSubcoreMesh(
        core_axis_name="c", subcore_axis_name="s",
        num_cores=NC, num_subcores=NS)
    out = pl.kernel(
        _body,
        mesh=mesh,
        out_type=jax.ShapeDtypeStruct((BATCH, 1, D_MODEL), jnp.float32),
        scratch_types=[
            pltpu.VMEM((BPW,), jnp.int32),
            pltpu.VMEM((L,), jnp.int32),
            pltpu.VMEM((BPW, D_MODEL), jnp.float32),
            pltpu.VMEM((1, D_MODEL), jnp.float32),
            pltpu.SemaphoreType.DMA,
            pltpu.SemaphoreType.DMA,
        ],
        name="speech_embedding_sc",
    )(next_token.reshape(NW, BPW), idx, speech_emb, pos_emb)
    return out
